# Initial kernel scaffold; baseline (speedup 1.0000x reference)
import jax
import jax.numpy as jnp
from jax.experimental import pallas as pl


def kernel(values, weights):
    raise NotImplementedError("write your pallas kernel here")


# speedup vs baseline: 12.5529x; 12.5529x over previous
import jax
import jax.numpy as jnp
from jax.experimental import pallas as pl
from jax.experimental.pallas import tpu as pltpu

_B, _C, _T, _H, _W = 4, 32, 16, 112, 112
_KT = _KS = 3
_THRESHOLD = 0.5
_LO, _HI = 0.0, 1.0


def _tv_loss_kernel(x_ref, out_ref):
    x = x_ref[0]
    f32 = jnp.float32

    zw = jnp.zeros((_T, _H, 1), f32)
    sw = (x
          + jnp.concatenate([zw, x[:, :, :-1]], axis=2)
          + jnp.concatenate([x[:, :, 1:], zw], axis=2))
    zh = jnp.zeros((_T, 1, _W), f32)
    sh = (sw
          + jnp.concatenate([zh, sw[:, :-1, :]], axis=1)
          + jnp.concatenate([sw[:, 1:, :], zh], axis=1))
    zt = jnp.zeros((1, _H, _W), f32)
    st = (sh
          + jnp.concatenate([zt, sh[:-1, :, :]], axis=0)
          + jnp.concatenate([sh[1:, :, :], zt], axis=0))

    soft = st * f32(1.0 / (_KT * _KS * _KS))
    trg = jnp.where(soft < _THRESHOLD, f32(_LO), f32(_HI))
    out_ref[...] = jnp.sum(jnp.abs(x - trg), keepdims=True)


def kernel(values, weights):
    del weights
    x = values.reshape(_B * _C, _T, _H, _W)
    partials = pl.pallas_call(
        _tv_loss_kernel,
        grid=(_B * _C,),
        in_specs=[pl.BlockSpec((1, _T, _H, _W), lambda i: (i, 0, 0, 0))],
        out_specs=pl.BlockSpec((1, 1, 1), lambda i: (i, 0, 0)),
        out_shape=jax.ShapeDtypeStruct((_B * _C, 1, 1), jnp.float32),
        compiler_params=pltpu.CompilerParams(
            dimension_semantics=("parallel",),
        ),
    )(x)
    return jnp.sum(partials) / jnp.float32(_B * _C * _T * _H * _W)


# speedup vs baseline: 13.2393x; 1.0547x over previous
import jax
import jax.numpy as jnp
from jax.experimental import pallas as pl
from jax.experimental.pallas import tpu as pltpu

_B, _C, _T, _H, _W = 4, 32, 16, 112, 112
_KT = _KS = 3
_SUM_THRESHOLD = 0.5 * (_KT * _KS * _KS)
_LO, _HI = 0.0, 1.0


def _hw_sum(xt):
    f32 = jnp.float32
    zw = jnp.zeros((_H, 1), f32)
    sw = (xt
          + jnp.concatenate([zw, xt[:, :-1]], axis=1)
          + jnp.concatenate([xt[:, 1:], zw], axis=1))
    zh = jnp.zeros((1, _W), f32)
    return (sw
            + jnp.concatenate([zh, sw[:-1, :]], axis=0)
            + jnp.concatenate([sw[1:, :], zh], axis=0))


def _tv_loss_kernel(x_ref, out_ref):
    def contrib(t, st):
        xt = x_ref[0, t]
        trg = jnp.where(st < _SUM_THRESHOLD, jnp.float32(_LO), jnp.float32(_HI))
        return jnp.abs(xt - trg)

    a = _hw_sum(x_ref[0, 0])
    b = _hw_sum(x_ref[0, 1])
    acc = contrib(0, a + b)
    for t in range(1, _T - 1):
        c = _hw_sum(x_ref[0, t + 1])
        acc = acc + contrib(t, a + b + c)
        a, b = b, c
    acc = acc + contrib(_T - 1, a + b)
    out_ref[...] = jnp.sum(acc, keepdims=True).reshape(1, 1, 1)


def kernel(values, weights):
    del weights
    x = values.reshape(_B * _C, _T, _H, _W)
    partials = pl.pallas_call(
        _tv_loss_kernel,
        grid=(_B * _C,),
        in_specs=[pl.BlockSpec((1, _T, _H, _W), lambda i: (i, 0, 0, 0))],
        out_specs=pl.BlockSpec((1, 1, 1), lambda i: (i, 0, 0)),
        out_shape=jax.ShapeDtypeStruct((_B * _C, 1, 1), jnp.float32),
        compiler_params=pltpu.CompilerParams(
            dimension_semantics=("parallel",),
        ),
    )(x)
    return jnp.sum(partials) / jnp.float32(_B * _C * _T * _H * _W)


# speedup vs baseline: 21.1371x; 1.5965x over previous
import jax
import jax.numpy as jnp
from jax.experimental import pallas as pl
from jax.experimental.pallas import tpu as pltpu

_B, _C, _T, _H, _W = 4, 32, 16, 112, 112
_KT = _KS = 3
_SUM_THRESHOLD = 0.5 * (_KT * _KS * _KS)
_LO, _HI = 0.0, 1.0


def _hw_sum(xt):
    f32 = jnp.float32
    zw = jnp.zeros((_H, 1), f32)
    sw = (xt
          + jnp.concatenate([zw, xt[:, :-1]], axis=1)
          + jnp.concatenate([xt[:, 1:], zw], axis=1))
    zh = jnp.zeros((1, _W), f32)
    return (sw
            + jnp.concatenate([zh, sw[:-1, :]], axis=0)
            + jnp.concatenate([sw[1:, :], zh], axis=0))


_C_BLK = 4


def _tv_loss_kernel(x_ref, out_ref):
    def contrib(c, t, st):
        xt = x_ref[c, t]
        trg = jnp.where(st < _SUM_THRESHOLD, jnp.float32(_LO), jnp.float32(_HI))
        return jnp.abs(xt - trg)

    acc = jnp.zeros((_H, _W), jnp.float32)
    for c in range(_C_BLK):
        a = _hw_sum(x_ref[c, 0])
        b = _hw_sum(x_ref[c, 1])
        acc = acc + contrib(c, 0, a + b)
        for t in range(1, _T - 1):
            nxt = _hw_sum(x_ref[c, t + 1])
            acc = acc + contrib(c, t, a + b + nxt)
            a, b = b, nxt
        acc = acc + contrib(c, _T - 1, a + b)
    out_ref[...] = jnp.sum(acc, keepdims=True).reshape(1, 1, 1)


def kernel(values, weights):
    del weights
    n_blocks = (_B * _C) // _C_BLK
    x = values.reshape(_B * _C, _T, _H, _W)
    partials = pl.pallas_call(
        _tv_loss_kernel,
        grid=(n_blocks,),
        in_specs=[pl.BlockSpec((_C_BLK, _T, _H, _W), lambda i: (i, 0, 0, 0))],
        out_specs=pl.BlockSpec((1, 1, 1), lambda i: (i, 0, 0)),
        out_shape=jax.ShapeDtypeStruct((n_blocks, 1, 1), jnp.float32),
        compiler_params=pltpu.CompilerParams(
            dimension_semantics=("parallel",),
        ),
    )(x)
    return jnp.sum(partials) / jnp.float32(_B * _C * _T * _H * _W)


# speedup vs baseline: 21.5871x; 1.0213x over previous
import jax
import jax.numpy as jnp
from jax.experimental import pallas as pl
from jax.experimental.pallas import tpu as pltpu

_B, _C, _T, _H, _W = 4, 32, 16, 112, 112
_KT = _KS = 3
_SUM_THRESHOLD = 0.5 * (_KT * _KS * _KS)
_LO, _HI = 0.0, 1.0


def _hw_sum(xt):
    f32 = jnp.float32
    zw = jnp.zeros((_H, 1), f32)
    sw = (xt
          + jnp.concatenate([zw, xt[:, :-1]], axis=1)
          + jnp.concatenate([xt[:, 1:], zw], axis=1))
    zh = jnp.zeros((1, _W), f32)
    return (sw
            + jnp.concatenate([zh, sw[:-1, :]], axis=0)
            + jnp.concatenate([sw[1:, :], zh], axis=0))


_C_BLK = 8


def _tv_loss_kernel(x_ref, out_ref):
    def contrib(c, t, st):
        xt = x_ref[c, t]
        trg = jnp.where(st < _SUM_THRESHOLD, jnp.float32(_LO), jnp.float32(_HI))
        return jnp.abs(xt - trg)

    acc = jnp.zeros((_H, _W), jnp.float32)
    for c in range(_C_BLK):
        a = _hw_sum(x_ref[c, 0])
        b = _hw_sum(x_ref[c, 1])
        acc = acc + contrib(c, 0, a + b)
        for t in range(1, _T - 1):
            nxt = _hw_sum(x_ref[c, t + 1])
            acc = acc + contrib(c, t, a + b + nxt)
            a, b = b, nxt
        acc = acc + contrib(c, _T - 1, a + b)
    out_ref[...] = jnp.sum(acc, keepdims=True).reshape(1, 1, 1)


def kernel(values, weights):
    del weights
    n_blocks = (_B * _C) // _C_BLK
    x = values.reshape(_B * _C, _T, _H, _W)
    partials = pl.pallas_call(
        _tv_loss_kernel,
        grid=(n_blocks,),
        in_specs=[pl.BlockSpec((_C_BLK, _T, _H, _W), lambda i: (i, 0, 0, 0))],
        out_specs=pl.BlockSpec((1, 1, 1), lambda i: (i, 0, 0)),
        out_shape=jax.ShapeDtypeStruct((n_blocks, 1, 1), jnp.float32),
        compiler_params=pltpu.CompilerParams(
            dimension_semantics=("parallel",),
        ),
    )(x)
    return jnp.sum(partials) / jnp.float32(_B * _C * _T * _H * _W)
